# Initial kernel scaffold; baseline (speedup 1.0000x reference)
import jax
import jax.numpy as jnp
from jax.experimental import pallas as pl


def kernel(input_ids, text_table, codebook, W_proj, b_proj):
    raise NotImplementedError("write your pallas kernel here")


# speedup vs baseline: 3.3784x; 3.3784x over previous
import functools

import jax
import jax.numpy as jnp
from jax import lax
from jax.experimental import pallas as pl
from jax.experimental.pallas import tpu as pltpu
from jax.experimental.pallas import tpu_sc as plsc

_CODEBOOK_K = 8192
_CODE_OFFSET = 40000
_ID_BOV = 49000
_ID_BOA = 49002
_N_CODES = 32
_WIN = 40


def _sc_gather(ids_flat, table):
  n = ids_flat.shape[0]
  d = table.shape[1]
  info = plsc.get_sparse_core_info()
  nw = info.num_cores * info.num_subcores
  per_w = n // nw
  ch = 8
  n_ch = per_w // ch
  mesh = plsc.VectorSubcoreMesh(core_axis_name="c", subcore_axis_name="s")

  @functools.partial(
      pl.kernel,
      mesh=mesh,
      out_type=jax.ShapeDtypeStruct((n, d), jnp.float32),
      scratch_types=[
          pltpu.VMEM((per_w,), jnp.int32),
          pltpu.VMEM((ch, d), jnp.float32),
          pltpu.VMEM((ch, d), jnp.float32),
          pltpu.SemaphoreType.DMA,
          pltpu.SemaphoreType.DMA,
          pltpu.SemaphoreType.DMA,
          pltpu.SemaphoreType.DMA,
      ],
  )
  def gather_kernel(ids_hbm, table_hbm, out_hbm, idx_v, rows0, rows1,
                    sg0, sg1, ss0, ss1):
    wid = lax.axis_index("s") * info.num_cores + lax.axis_index("c")
    base = wid * per_w
    pltpu.sync_copy(ids_hbm.at[pl.ds(base, per_w)], idx_v)
    bufs = (rows0, rows1)
    sgs = (sg0, sg1)
    sss = (ss0, ss1)

    def g_start(c, b):
      pltpu.async_copy(table_hbm.at[idx_v.at[pl.ds(c * ch, ch)]], bufs[b],
                       sgs[b])

    def g_wait(b):
      pltpu.make_async_copy(table_hbm.at[idx_v.at[pl.ds(0, ch)]], bufs[b],
                            sgs[b]).wait()

    def s_start(c, b):
      pltpu.async_copy(bufs[b], out_hbm.at[pl.ds(base + c * ch, ch)], sss[b])

    def s_wait(c, b):
      pltpu.make_async_copy(bufs[b], out_hbm.at[pl.ds(base + c * ch, ch)],
                            sss[b]).wait()

    g_start(0, 0)
    g_start(1, 1)

    def body(i, carry):
      for b in range(2):
        c = 2 * i + b
        g_wait(b)
        s_start(c, b)
        s_wait(c, b)
        g_start(c + 2, b)
      return carry

    lax.fori_loop(0, n_ch // 2 - 1, body, 0)
    for b in range(2):
      c = n_ch - 2 + b
      g_wait(b)
      s_start(c, b)
      s_wait(c, b)

  return gather_kernel(ids_flat, table)


def _proj_kernel(ids3_ref, cb_ref, w_ref, b_ref, proj_ref, pos_ref):
  b_batches, sub, lane = ids3_ref.shape
  l_seq = sub * lane
  flat_pos = (lax.broadcasted_iota(jnp.int32, (sub, lane), 0) * lane
              + lax.broadcasted_iota(jnp.int32, (sub, lane), 1))
  for b in range(b_batches):
    row = ids3_ref[b]
    p_bov = jnp.min(jnp.where(row == _ID_BOV, flat_pos, l_seq))
    p_boa = jnp.min(jnp.where(row == _ID_BOA, flat_pos, l_seq))
    lane_iota = lax.broadcasted_iota(jnp.int32, (1, 128), 1)
    pos_ref[pl.ds(b, 1), :] = jnp.where(lane_iota == 0, p_bov,
                                        jnp.where(lane_iota == 1, p_boa, 0))
    jg = lax.broadcasted_iota(jnp.int32, (2 * _N_CODES, 1), 0)
    t = jnp.where(jg < _N_CODES, p_bov + 1 + jg, p_boa + 1 + jg - _N_CODES)
    rmask = (lax.broadcasted_iota(jnp.int32, (2 * _N_CODES, sub), 1)
             == t // lane).astype(jnp.float32)
    cmask = (lax.broadcasted_iota(jnp.int32, (2 * _N_CODES, lane), 1)
             == t % lane).astype(jnp.float32)
    row_f = row.astype(jnp.float32)
    picked = jnp.dot(rmask, row_f, preferred_element_type=jnp.float32,
                     precision=lax.Precision.HIGHEST)
    codes = jnp.sum(picked * cmask, axis=1, keepdims=True).astype(jnp.int32)
    codes = codes - _CODE_OFFSET
    onehot = (lax.broadcasted_iota(jnp.int32, (2 * _N_CODES, _CODEBOOK_K), 1)
              == codes).astype(jnp.float32)
    emb = jnp.dot(onehot, cb_ref[...], preferred_element_type=jnp.float32,
                  precision=lax.Precision.HIGHEST)
    proj_ref[b] = (jnp.dot(emb, w_ref[...], preferred_element_type=jnp.float32,
                           precision=lax.Precision.HIGHEST) + b_ref[...])


def _tc_proj(input_ids, codebook, w_proj, b_proj):
  bsz, l_seq = input_ids.shape
  d = w_proj.shape[1]
  ids3 = input_ids.reshape(bsz, l_seq // 128, 128)
  return pl.pallas_call(
      _proj_kernel,
      out_shape=(
          jax.ShapeDtypeStruct((bsz, 2 * _N_CODES, d), jnp.float32),
          jax.ShapeDtypeStruct((bsz, 128), jnp.int32),
      ),
      in_specs=[
          pl.BlockSpec(memory_space=pltpu.VMEM),
          pl.BlockSpec(memory_space=pltpu.VMEM),
          pl.BlockSpec(memory_space=pltpu.VMEM),
          pl.BlockSpec(memory_space=pltpu.VMEM),
      ],
      out_specs=(
          pl.BlockSpec(memory_space=pltpu.VMEM),
          pl.BlockSpec(memory_space=pltpu.VMEM),
      ),
  )(ids3, codebook, w_proj, b_proj.reshape(1, d))


def _merge_kernel(pos_ref, proj_ref, out0_ref, out_ref, buf_v, sem):
  del out0_ref
  b_batches = proj_ref.shape[0]
  n_win = 2 * b_batches
  rowi = lax.broadcasted_iota(jnp.int32, (_WIN, _N_CODES), 0)
  colj = lax.broadcasted_iota(jnp.int32, (_WIN, _N_CODES), 1)

  def win_params(k):
    b, g = k // 2, k % 2
    start = pos_ref[b, g] + 1
    a = pl.multiple_of((start // 8) * 8, 8)
    return b, g, a, start - a

  reads = []
  for k in range(n_win):
    b, g, a, o = win_params(k)
    cp = pltpu.make_async_copy(out_ref.at[b, pl.ds(a, _WIN), :],
                               buf_v.at[k], sem)
    cp.start()
    reads.append(cp)
  for k in range(n_win):
    reads[k].wait()
  writes = []
  for k in range(n_win):
    b, g, a, o = win_params(k)
    perm = (colj == rowi - o).astype(jnp.float32)
    shifted = jnp.dot(perm, proj_ref[b, pl.ds(g * _N_CODES, _N_CODES), :],
                      preferred_element_type=jnp.float32,
                      precision=lax.Precision.HIGHEST)
    sel = (rowi[:, :1] >= o) & (rowi[:, :1] < o + _N_CODES)
    buf_v[k] = jnp.where(sel, shifted, buf_v[k])
    cp = pltpu.make_async_copy(buf_v.at[k], out_ref.at[b, pl.ds(a, _WIN), :],
                               sem)
    cp.start()
    writes.append(cp)
  for k in range(n_win):
    writes[k].wait()


def _tc_merge(out, pos, proj):
  bsz, l_seq, d = out.shape
  return pl.pallas_call(
      _merge_kernel,
      out_shape=jax.ShapeDtypeStruct((bsz, l_seq, d), jnp.float32),
      in_specs=[
          pl.BlockSpec(memory_space=pltpu.SMEM),
          pl.BlockSpec(memory_space=pltpu.VMEM),
          pl.BlockSpec(memory_space=pl.ANY),
      ],
      out_specs=pl.BlockSpec(memory_space=pl.ANY),
      scratch_shapes=[
          pltpu.VMEM((2 * bsz, _WIN, d), jnp.float32),
          pltpu.SemaphoreType.DMA,
      ],
      input_output_aliases={2: 0},
  )(pos, proj, out)


def kernel(input_ids, text_table, codebook, W_proj, b_proj):
  bsz, l_seq = input_ids.shape
  d = text_table.shape[1]
  proj, pos = _tc_proj(input_ids, codebook, W_proj, b_proj)
  ids_flat = input_ids.reshape(bsz * l_seq)
  out_flat = _sc_gather(ids_flat, text_table)
  out = out_flat.reshape(bsz, l_seq, d)
  return _tc_merge(out, pos[:, :2], proj)
